# Initial kernel scaffold; baseline (speedup 1.0000x reference)
import jax
import jax.numpy as jnp
from jax.experimental import pallas as pl


def kernel(edge_features, graph_embedding, selected_commodity, candidate_paths, path_mask, W1, b1, W2, b2):
    raise NotImplementedError("write your pallas kernel here")


# speedup vs baseline: 1.2347x; 1.2347x over previous
import functools

import jax
import jax.numpy as jnp
from jax import lax
from jax.experimental import pallas as pl
from jax.experimental.pallas import tpu as pltpu
from jax.experimental.pallas import tpu_sc as plsc

_B, _N, _C, _H, _P, _L = 16, 32, 8, 256, 64, 8
_NPATH = _B * _P
_EDGES = _L - 1
_NW = 32
_PPW = _NPATH // _NW
_EPW = _PPW * _EDGES


def _sc_body(table_hbm, ids_hbm, out_hbm, ids_v, rows_v, out_v, sem):
    wid = lax.axis_index("s") * 2 + lax.axis_index("c")
    pltpu.sync_copy(ids_hbm.at[wid], ids_v)
    cp0 = pltpu.async_copy(table_hbm.at[ids_v.at[0]], rows_v.at[pl.ds(0, 112)], sem)
    cp1 = pltpu.async_copy(table_hbm.at[ids_v.at[1]], rows_v.at[pl.ds(112, 112)], sem)
    cp0.wait()
    cp1.wait()

    def mean_one_path(p, carry):
        for h in range(_H // 16):
            acc = rows_v[p * _EDGES, pl.ds(16 * h, 16)]
            for e in range(1, _EDGES):
                acc = acc + rows_v[p * _EDGES + e, pl.ds(16 * h, 16)]
            out_v[p, pl.ds(16 * h, 16)] = acc * (1.0 / _EDGES)
        return carry

    lax.fori_loop(0, _PPW, mean_one_path, 0)
    pltpu.sync_copy(out_v, out_hbm.at[pl.ds(wid * _PPW, _PPW)])


@functools.cache
def _sc_gather_mean():
    return pl.kernel(
        _sc_body,
        mesh=plsc.VectorSubcoreMesh(core_axis_name="c", subcore_axis_name="s"),
        out_type=jax.ShapeDtypeStruct((_NPATH, _H), jnp.float32),
        scratch_types=[
            pltpu.VMEM((2, _EPW // 2), jnp.int32),
            pltpu.VMEM((_EPW, _H), jnp.float32),
            pltpu.VMEM((_PPW, _H), jnp.float32),
            pltpu.SemaphoreType.DMA,
        ],
    )


def _tc_body(pf_ref, g_ref, w1_ref, b1_ref, w2_ref, b2_ref, mask_ref,
             probs_ref, logp_ref, ent_ref):
    pf = pf_ref[...]
    h = jnp.dot(pf, w1_ref[0:_H, :], preferred_element_type=jnp.float32)
    hg = jnp.dot(g_ref[...], w1_ref[_H:2 * _H, :],
                 preferred_element_type=jnp.float32)
    hg = hg + b1_ref[...]
    h = h.reshape(_B, _P, 128) + hg[:, None, :]
    h = jnp.maximum(h, 0.0)
    w2 = w2_ref[...]
    scores = jnp.sum(h * w2[None, :, :], axis=-1) + b2_ref[0, 0]
    mask = mask_ref[...] != 0
    scores = jnp.where(mask, scores, -jnp.inf)
    m = jnp.max(scores, axis=-1, keepdims=True)
    ex = jnp.exp(scores - m)
    s = jnp.sum(ex, axis=-1, keepdims=True)
    probs = ex / s
    logp = (scores - m) - jnp.log(s)
    logp_safe = jnp.where(mask, logp, 0.0)
    ent = -jnp.sum(probs * logp_safe, axis=-1, keepdims=True)
    probs_ref[...] = probs
    logp_ref[...] = logp
    ent_ref[...] = ent


_tc_mlp_softmax = pl.pallas_call(
    _tc_body,
    out_shape=(
        jax.ShapeDtypeStruct((_B, _P), jnp.float32),
        jax.ShapeDtypeStruct((_B, _P), jnp.float32),
        jax.ShapeDtypeStruct((_B, 1), jnp.float32),
    ),
)


def kernel(edge_features, graph_embedding, selected_commodity, candidate_paths,
           path_mask, W1, b1, W2, b2):
    table = edge_features.reshape(_B * _N * _N * _C, _H)
    cp = candidate_paths.astype(jnp.int32)
    u = cp[:, :, :-1]
    v = cp[:, :, 1:]
    b_idx = jnp.arange(_B, dtype=jnp.int32)[:, None, None]
    c_idx = selected_commodity.astype(jnp.int32)[:, None, None]
    ids = (((b_idx * _N + u) * _N + v) * _C + c_idx).reshape(_NW, 2, _EPW // 2)

    path_feat = _sc_gather_mean()(table, ids)

    probs, logp, ent = _tc_mlp_softmax(
        path_feat,
        graph_embedding,
        W1,
        b1.reshape(1, 128),
        W2.reshape(1, 128),
        b2.reshape(1, 1),
        path_mask.astype(jnp.int32),
    )
    return (probs, logp, ent.reshape(_B))
